# Initial kernel scaffold; baseline (speedup 1.0000x reference)
import jax
import jax.numpy as jnp
from jax.experimental import pallas as pl


def kernel(x, edge_index, W, att_src, att_dst, bias):
    raise NotImplementedError("write your pallas kernel here")


# speedup vs baseline: 17.8801x; 17.8801x over previous
import functools

import jax
import jax.numpy as jnp
from jax import lax
from jax.experimental import pallas as pl
from jax.experimental.pallas import tpu as pltpu
from jax.experimental.pallas import tpu_sc as plsc

_N = 10000
_E = 320000
_D = 128
_NPAD = 10240
_NW = 32
_EPT = _E // _NW
_B1 = 400
_C = 80
_SL = _NPAD // 16
_ZR = 40

_mesh = plsc.VectorSubcoreMesh(core_axis_name="c", subcore_axis_name="s")


_BP = 512


def _proj_body(x_ref, w_ref, as_ref, ad_ref, xw_ref, asrc_ref, adst_ref):
    xw = jnp.dot(x_ref[...], w_ref[...], preferred_element_type=jnp.float32)
    xw_ref[...] = xw
    a_s = jnp.sum(xw * as_ref[...], axis=1)
    a_d = jnp.sum(xw * ad_ref[...], axis=1)
    asrc_ref[...] = jnp.broadcast_to(a_s[None, :], (8, _BP))
    adst_ref[...] = jnp.broadcast_to(a_d[None, :], (8, _BP))


def _tc_proj(x, w, att_s, att_d):
    grid = _NPAD // _BP
    return pl.pallas_call(
        _proj_body,
        grid=(grid,),
        in_specs=[
            pl.BlockSpec((_BP, _D), lambda i: (i, 0)),
            pl.BlockSpec((_D, _D), lambda i: (0, 0)),
            pl.BlockSpec((1, _D), lambda i: (0, 0)),
            pl.BlockSpec((1, _D), lambda i: (0, 0)),
        ],
        out_specs=[
            pl.BlockSpec((_BP, _D), lambda i: (i, 0)),
            pl.BlockSpec((8, _BP), lambda i: (0, i)),
            pl.BlockSpec((8, _BP), lambda i: (0, i)),
        ],
        out_shape=[
            jax.ShapeDtypeStruct((_NPAD, _D), jnp.float32),
            jax.ShapeDtypeStruct((8, _NPAD), jnp.float32),
            jax.ShapeDtypeStruct((8, _NPAD), jnp.float32),
        ],
    )(x, w, att_s, att_d)


@functools.partial(
    pl.kernel,
    mesh=_mesh,
    compiler_params=pltpu.CompilerParams(needs_layout_passes=False),
    out_type=[
        jax.ShapeDtypeStruct((_E,), jnp.float32),
        jax.ShapeDtypeStruct((2, _NPAD), jnp.float32),
    ],
    scratch_types=[
        pltpu.VMEM((_NPAD,), jnp.float32),
        pltpu.VMEM((_NPAD,), jnp.float32),
        pltpu.VMEM((_NPAD,), jnp.float32),
        pltpu.VMEM((_B1,), jnp.int32),
        pltpu.VMEM((_B1,), jnp.int32),
        pltpu.VMEM((_B1,), jnp.float32),
        pltpu.VMEM_SHARED((16, _NPAD), jnp.float32),
        pltpu.VMEM((16, _SL), jnp.float32),
    ],
)
def _sc_alpha_amax(src_h, dst_h, asrc_h, adst_h, alpha_h, amax_h,
                   asrc_v, adst_v, amax_v, src_v, dst_v, al_v, stage_sh, red_v):
    cid = lax.axis_index("c")
    sid = lax.axis_index("s")
    wid = cid * 16 + sid

    pltpu.sync_copy(asrc_h, asrc_v)
    pltpu.sync_copy(adst_h, adst_v)

    neg_inf = jnp.full((16,), -jnp.inf, jnp.float32)

    def _init(i, carry):
        amax_v[pl.ds(i * 16, 16)] = neg_inf
        return carry

    lax.fori_loop(0, _NPAD // 16, _init, 0)

    ebase = wid * _EPT

    def _blk(b, carry):
        off = ebase + b * _B1
        pltpu.sync_copy(src_h.at[pl.ds(off, _B1)], src_v)
        pltpu.sync_copy(dst_h.at[pl.ds(off, _B1)], dst_v)

        def _grp(g, c2):
            s = src_v[pl.ds(g * 16, 16)]
            d = dst_v[pl.ds(g * 16, 16)]
            al = plsc.load_gather(asrc_v, [s]) + plsc.load_gather(adst_v, [d])
            al = jnp.where(al >= 0.0, al, al * 0.2)
            al_v[pl.ds(g * 16, 16)] = al

            cur = plsc.load_gather(amax_v, [d])
            need = al > cur
            plsc.store_scatter(amax_v, [d], al, mask=need)
            chk = plsc.load_gather(amax_v, [d])
            still = jnp.where(jnp.logical_and(need, chk < al), 1, 0)

            def _cond(st):
                return jnp.max(st) > 0

            def _body(st):
                m = st > 0
                plsc.store_scatter(amax_v, [d], al, mask=m)
                c = plsc.load_gather(amax_v, [d])
                return jnp.where(jnp.logical_and(m, c < al), 1, 0)

            lax.while_loop(_cond, _body, still)
            return c2

        lax.fori_loop(0, _B1 // 16, _grp, 0)
        pltpu.sync_copy(al_v, alpha_h.at[pl.ds(off, _B1)])
        return carry

    lax.fori_loop(0, _EPT // _B1, _blk, 0)

    pltpu.sync_copy(amax_v, stage_sh.at[sid])
    plsc.subcore_barrier()
    colbase = sid * _SL
    for r in range(16):
        pltpu.sync_copy(stage_sh.at[r, pl.ds(colbase, _SL)], red_v.at[r])

    def _red(i, carry):
        m = red_v[0, pl.ds(i * 16, 16)]
        for r in range(1, 16):
            m = jnp.maximum(m, red_v[r, pl.ds(i * 16, 16)])
        amax_v[pl.ds(i * 16, 16)] = m
        return carry

    lax.fori_loop(0, _SL // 16, _red, 0)
    pltpu.sync_copy(amax_v.at[pl.ds(0, _SL)], amax_h.at[cid, pl.ds(colbase, _SL)])


@functools.partial(
    pl.kernel,
    mesh=_mesh,
    compiler_params=pltpu.CompilerParams(needs_layout_passes=False),
    out_type=[
        jax.ShapeDtypeStruct((2, _NPAD, _D), jnp.float32),
        jax.ShapeDtypeStruct((2, _NPAD), jnp.float32),
    ],
    scratch_types=[
        pltpu.VMEM((2, _NPAD), jnp.float32),
        pltpu.VMEM((_NPAD,), jnp.float32),
        pltpu.VMEM((_C,), jnp.int32),
        pltpu.VMEM((_C,), jnp.int32),
        pltpu.VMEM((_C,), jnp.float32),
        pltpu.VMEM((_C,), jnp.float32),
        pltpu.VMEM((_C, _D), jnp.float32),
        pltpu.VMEM((_ZR, _D), jnp.float32),
        pltpu.VMEM((_SL,), jnp.float32),
        pltpu.VMEM_SHARED((_NPAD, _D), jnp.float32),
        pltpu.VMEM_SHARED((_NPAD,), jnp.float32),
        pltpu.SemaphoreType.DMA,
    ],
)
def _sc_aggregate(src_h, dst_h, alpha_h, amax_h, xw_h, acc_out, den_out,
                  amx2_v, amf_v, src_v, dst_v, al_v, p_v, rows_v, zrow_v,
                  zden_v, acc_sh, den_sh, sem):
    cid = lax.axis_index("c")
    sid = lax.axis_index("s")
    wid = cid * 16 + sid
    rowbase = sid * _SL

    pltpu.sync_copy(amax_h, amx2_v)

    def _amf(i, carry):
        amf_v[pl.ds(i * 16, 16)] = jnp.maximum(
            amx2_v[0, pl.ds(i * 16, 16)], amx2_v[1, pl.ds(i * 16, 16)])
        return carry

    lax.fori_loop(0, _NPAD // 16, _amf, 0)

    zero16 = jnp.zeros((16,), jnp.float32)

    def _zr(i, carry):
        for j in range(_D // 16):
            zrow_v[i, pl.ds(j * 16, 16)] = zero16
        return carry

    lax.fori_loop(0, _ZR, _zr, 0)

    def _zd(i, carry):
        zden_v[pl.ds(i * 16, 16)] = zero16
        return carry

    lax.fori_loop(0, _SL // 16, _zd, 0)

    for k in range(_SL // _ZR):
        pltpu.sync_copy(zrow_v, acc_sh.at[pl.ds(rowbase + k * _ZR, _ZR)])
    pltpu.sync_copy(zden_v, den_sh.at[pl.ds(rowbase, _SL)])
    plsc.subcore_barrier()

    ebase = wid * _EPT

    def _blk(b, carry):
        off = ebase + b * _C
        pltpu.sync_copy(src_h.at[pl.ds(off, _C)], src_v)
        pltpu.sync_copy(dst_h.at[pl.ds(off, _C)], dst_v)
        pltpu.sync_copy(alpha_h.at[pl.ds(off, _C)], al_v)
        pltpu.async_copy(xw_h.at[src_v], rows_v, sem).wait()

        def _grp(g, c2):
            d = dst_v[pl.ds(g * 16, 16)]
            am = plsc.load_gather(amf_v, [d])
            p_v[pl.ds(g * 16, 16)] = jnp.exp(al_v[pl.ds(g * 16, 16)] - am)
            return c2

        lax.fori_loop(0, _C // 16, _grp, 0)

        def _scale(i, c2):
            pi = plsc.load_gather(p_v, [jnp.full((16,), i, jnp.int32)])
            for j in range(_D // 16):
                rows_v[i, pl.ds(j * 16, 16)] = rows_v[i, pl.ds(j * 16, 16)] * pi
            return c2

        lax.fori_loop(0, _C, _scale, 0)

        pltpu.sync_copy(rows_v, acc_sh.at[dst_v], add=True)
        pltpu.sync_copy(p_v, den_sh.at[dst_v], add=True)
        return carry

    lax.fori_loop(0, _EPT // _C, _blk, 0)
    plsc.subcore_barrier()

    pltpu.sync_copy(acc_sh.at[pl.ds(rowbase, _SL)],
                    acc_out.at[cid, pl.ds(rowbase, _SL)])
    pltpu.sync_copy(den_sh.at[pl.ds(rowbase, _SL)],
                    den_out.at[cid, pl.ds(rowbase, _SL)])


def _comb_body(acc_ref, den_ref, bias_ref, out_ref):
    i = pl.program_id(0)
    a = acc_ref[0] + acc_ref[1]
    off = pl.multiple_of(i * _BP, 128)
    den = den_ref[:, pl.ds(off, _BP)]
    dsum = den[0] + den[1] + jnp.float32(1e-16)
    out_ref[...] = a / dsum[:, None] + bias_ref[...]


def _tc_combine(acc2, den2, bias):
    grid = _NPAD // _BP
    return pl.pallas_call(
        _comb_body,
        grid=(grid,),
        in_specs=[
            pl.BlockSpec((2, _BP, _D), lambda i: (0, i, 0)),
            pl.BlockSpec((2, _NPAD), lambda i: (0, 0)),
            pl.BlockSpec((1, _D), lambda i: (0, 0)),
        ],
        out_specs=pl.BlockSpec((_BP, _D), lambda i: (i, 0)),
        out_shape=jax.ShapeDtypeStruct((_NPAD, _D), jnp.float32),
    )(acc2, den2, bias)


def kernel(x, edge_index, W, att_src, att_dst, bias):
    att_s = att_src.reshape(1, _D)
    att_d = att_dst.reshape(1, _D)
    xp = jnp.pad(x, ((0, _NPAD - _N), (0, 0)))
    xw, asrc8, adst8 = _tc_proj(xp, W, att_s, att_d)
    src = edge_index[0]
    dst = edge_index[1]
    alpha, amax2 = _sc_alpha_amax(src, dst, asrc8[0], adst8[0])
    acc2, den2 = _sc_aggregate(src, dst, alpha, amax2, xw)
    out = _tc_combine(acc2, den2, bias.reshape(1, _D))
    return out[:_N]


# speedup vs baseline: 24.5511x; 1.3731x over previous
import functools

import jax
import jax.numpy as jnp
from jax import lax
from jax.experimental import pallas as pl
from jax.experimental.pallas import tpu as pltpu
from jax.experimental.pallas import tpu_sc as plsc

_N = 10000
_E = 320000
_D = 128
_NPAD = 10240
_NW = 32
_EPT = _E // _NW
_B1 = 400
_C = 80
_SL = _NPAD // 16
_CH = 1280

_mesh = plsc.VectorSubcoreMesh(core_axis_name="c", subcore_axis_name="s")


_BP = 512


def _proj_body(x_ref, w_ref, as_ref, ad_ref, xw_ref, asrc_ref, adst_ref):
    xw = jnp.dot(x_ref[...], w_ref[...], preferred_element_type=jnp.float32)
    xw_ref[...] = xw
    a_s = jnp.sum(xw * as_ref[...], axis=1)
    a_d = jnp.sum(xw * ad_ref[...], axis=1)
    asrc_ref[...] = jnp.broadcast_to(a_s[None, :], (8, _BP))
    adst_ref[...] = jnp.broadcast_to(a_d[None, :], (8, _BP))


def _tc_proj(x, w, att_s, att_d):
    grid = _NPAD // _BP
    return pl.pallas_call(
        _proj_body,
        grid=(grid,),
        in_specs=[
            pl.BlockSpec((_BP, _D), lambda i: (i, 0)),
            pl.BlockSpec((_D, _D), lambda i: (0, 0)),
            pl.BlockSpec((1, _D), lambda i: (0, 0)),
            pl.BlockSpec((1, _D), lambda i: (0, 0)),
        ],
        out_specs=[
            pl.BlockSpec((_BP, _D), lambda i: (i, 0)),
            pl.BlockSpec((8, _BP), lambda i: (0, i)),
            pl.BlockSpec((8, _BP), lambda i: (0, i)),
        ],
        out_shape=[
            jax.ShapeDtypeStruct((_NPAD, _D), jnp.float32),
            jax.ShapeDtypeStruct((8, _NPAD), jnp.float32),
            jax.ShapeDtypeStruct((8, _NPAD), jnp.float32),
        ],
    )(x, w, att_s, att_d)


@functools.partial(
    pl.kernel,
    mesh=_mesh,
    compiler_params=pltpu.CompilerParams(needs_layout_passes=False),
    out_type=[
        jax.ShapeDtypeStruct((_E,), jnp.float32),
        jax.ShapeDtypeStruct((2, _NPAD), jnp.float32),
    ],
    scratch_types=[
        pltpu.VMEM((_NPAD,), jnp.float32),
        pltpu.VMEM((_NPAD,), jnp.float32),
        pltpu.VMEM((_NPAD,), jnp.float32),
        pltpu.VMEM((_B1,), jnp.int32),
        pltpu.VMEM((_B1,), jnp.int32),
        pltpu.VMEM((_B1,), jnp.float32),
        pltpu.VMEM_SHARED((16, _NPAD), jnp.float32),
        pltpu.VMEM((16, _SL), jnp.float32),
    ],
)
def _sc_alpha_amax(src_h, dst_h, asrc_h, adst_h, alpha_h, amax_h,
                   asrc_v, adst_v, amax_v, src_v, dst_v, al_v, stage_sh, red_v):
    cid = lax.axis_index("c")
    sid = lax.axis_index("s")
    wid = cid * 16 + sid

    pltpu.sync_copy(asrc_h, asrc_v)
    pltpu.sync_copy(adst_h, adst_v)

    neg_inf = jnp.full((16,), -jnp.inf, jnp.float32)

    def _init(i, carry):
        amax_v[pl.ds(i * 16, 16)] = neg_inf
        return carry

    lax.fori_loop(0, _NPAD // 16, _init, 0)

    ebase = wid * _EPT

    def _blk(b, carry):
        off = ebase + b * _B1
        pltpu.sync_copy(src_h.at[pl.ds(off, _B1)], src_v)
        pltpu.sync_copy(dst_h.at[pl.ds(off, _B1)], dst_v)

        def _grp(g, c2):
            s = src_v[pl.ds(g * 16, 16)]
            d = dst_v[pl.ds(g * 16, 16)]
            al = plsc.load_gather(asrc_v, [s]) + plsc.load_gather(adst_v, [d])
            al = jnp.where(al >= 0.0, al, al * 0.2)
            al_v[pl.ds(g * 16, 16)] = al

            cur = plsc.load_gather(amax_v, [d])
            need = al > cur
            plsc.store_scatter(amax_v, [d], al, mask=need)
            chk = plsc.load_gather(amax_v, [d])
            still = jnp.where(jnp.logical_and(need, chk < al), 1, 0)

            def _cond(st):
                return jnp.max(st) > 0

            def _body(st):
                m = st > 0
                plsc.store_scatter(amax_v, [d], al, mask=m)
                c = plsc.load_gather(amax_v, [d])
                return jnp.where(jnp.logical_and(m, c < al), 1, 0)

            lax.while_loop(_cond, _body, still)
            return c2

        lax.fori_loop(0, _B1 // 16, _grp, 0)
        pltpu.sync_copy(al_v, alpha_h.at[pl.ds(off, _B1)])
        return carry

    lax.fori_loop(0, _EPT // _B1, _blk, 0)

    pltpu.sync_copy(amax_v, stage_sh.at[sid])
    plsc.subcore_barrier()
    colbase = sid * _SL
    for r in range(16):
        pltpu.sync_copy(stage_sh.at[r, pl.ds(colbase, _SL)], red_v.at[r])

    def _red(i, carry):
        m = red_v[0, pl.ds(i * 16, 16)]
        for r in range(1, 16):
            m = jnp.maximum(m, red_v[r, pl.ds(i * 16, 16)])
        amax_v[pl.ds(i * 16, 16)] = m
        return carry

    lax.fori_loop(0, _SL // 16, _red, 0)
    pltpu.sync_copy(amax_v.at[pl.ds(0, _SL)], amax_h.at[cid, pl.ds(colbase, _SL)])


@functools.partial(
    pl.kernel,
    mesh=_mesh,
    compiler_params=pltpu.CompilerParams(needs_layout_passes=False),
    out_type=[
        jax.ShapeDtypeStruct((2, _NPAD, _D), jnp.float32),
        jax.ShapeDtypeStruct((2, _NPAD), jnp.float32),
    ],
    scratch_types=[
        pltpu.VMEM((_NPAD,), jnp.float32),
        pltpu.VMEM((_CH,), jnp.float32),
        pltpu.VMEM((_CH,), jnp.float32),
        pltpu.VMEM((_C,), jnp.int32),
        pltpu.VMEM((_C,), jnp.int32),
        pltpu.VMEM((_C,), jnp.float32),
        pltpu.VMEM((_C,), jnp.float32),
        pltpu.VMEM((_C, _D), jnp.float32),
        pltpu.VMEM((_C,), jnp.int32),
        pltpu.VMEM((_C,), jnp.int32),
        pltpu.VMEM((_C,), jnp.float32),
        pltpu.VMEM((_C,), jnp.float32),
        pltpu.VMEM((_C, _D), jnp.float32),
        pltpu.VMEM_SHARED((_NPAD, _D), jnp.float32),
        pltpu.VMEM_SHARED((_NPAD,), jnp.float32),
        pltpu.SemaphoreType.DMA,
        pltpu.SemaphoreType.DMA,
    ],
)
def _sc_aggregate(src_h, dst_h, alpha_h, amax_h, xw_h, acc_out, den_out,
                  amf_v, tmp0_v, tmp1_v, src_v0, dst_v0, al_v0, p_v0, rows_v0,
                  src_v1, dst_v1, al_v1, p_v1, rows_v1, acc_sh, den_sh,
                  sem0, sem1):
    cid = lax.axis_index("c")
    sid = lax.axis_index("s")
    wid = cid * 16 + sid
    rowbase = sid * _SL

    def _amfc(c, carry):
        off = c * _CH
        pltpu.sync_copy(amax_h.at[0, pl.ds(off, _CH)], tmp0_v)
        pltpu.sync_copy(amax_h.at[1, pl.ds(off, _CH)], tmp1_v)

        def _m(i, c2):
            amf_v[pl.ds(off + i * 16, 16)] = jnp.maximum(
                tmp0_v[pl.ds(i * 16, 16)], tmp1_v[pl.ds(i * 16, 16)])
            return c2

        lax.fori_loop(0, _CH // 16, _m, 0)
        return carry

    lax.fori_loop(0, _NPAD // _CH, _amfc, 0)

    zero16 = jnp.zeros((16,), jnp.float32)

    def _zr(i, carry):
        for j in range(_D // 16):
            rows_v0[i, pl.ds(j * 16, 16)] = zero16
        return carry

    lax.fori_loop(0, _C, _zr, 0)

    def _zd(i, carry):
        p_v0[pl.ds(i * 16, 16)] = zero16
        return carry

    lax.fori_loop(0, _C // 16, _zd, 0)

    for k in range(_SL // _C):
        pltpu.sync_copy(rows_v0, acc_sh.at[pl.ds(rowbase + k * _C, _C)])
        pltpu.sync_copy(p_v0, den_sh.at[pl.ds(rowbase + k * _C, _C)])
    plsc.subcore_barrier()

    ebase = wid * _EPT
    bufs = (
        (src_v0, dst_v0, al_v0, p_v0, rows_v0, sem0),
        (src_v1, dst_v1, al_v1, p_v1, rows_v1, sem1),
    )

    def _load_idx(j, t):
        sv, dv, av, _, _, _ = bufs[t]
        off = ebase + j * _C
        pltpu.sync_copy(src_h.at[pl.ds(off, _C)], sv)
        pltpu.sync_copy(dst_h.at[pl.ds(off, _C)], dv)
        pltpu.sync_copy(alpha_h.at[pl.ds(off, _C)], av)

    def _start_gather(t):
        sv, _, _, _, rv, sm = bufs[t]
        pltpu.async_copy(xw_h.at[sv], rv, sm)

    def _wait_gather(t):
        sv, _, _, _, rv, sm = bufs[t]
        pltpu.make_async_copy(xw_h.at[sv], rv, sm).wait()

    def _process(t):
        _, dv, av, pv, rv, _ = bufs[t]

        def _grp(g, c2):
            d = dv[pl.ds(g * 16, 16)]
            am = plsc.load_gather(amf_v, [d])
            pv[pl.ds(g * 16, 16)] = jnp.exp(av[pl.ds(g * 16, 16)] - am)
            return c2

        lax.fori_loop(0, _C // 16, _grp, 0)

        @plsc.parallel_loop(0, _C, unroll=2)
        def _scale(i):
            pi = plsc.load_gather(pv, [jnp.full((16,), i, jnp.int32)])
            for j in range(_D // 16):
                rv[i, pl.ds(j * 16, 16)] = rv[i, pl.ds(j * 16, 16)] * pi

        pltpu.sync_copy(rv, acc_sh.at[dv], add=True)
        pltpu.sync_copy(pv, den_sh.at[dv], add=True)

    nblk = _EPT // _C
    _load_idx(0, 0)
    _start_gather(0)

    def _outer(b2, carry):
        j = 2 * b2
        _load_idx(j + 1, 1)
        _start_gather(1)
        _wait_gather(0)
        _process(0)
        _load_idx(j + 2, 0)
        _start_gather(0)
        _wait_gather(1)
        _process(1)
        return carry

    lax.fori_loop(0, nblk // 2, _outer, 0)
    _wait_gather(0)
    _process(0)
    plsc.subcore_barrier()

    pltpu.sync_copy(acc_sh.at[pl.ds(rowbase, _SL)],
                    acc_out.at[cid, pl.ds(rowbase, _SL)])
    pltpu.sync_copy(den_sh.at[pl.ds(rowbase, _SL)],
                    den_out.at[cid, pl.ds(rowbase, _SL)])


def _comb_body(acc_ref, den_ref, bias_ref, out_ref):
    i = pl.program_id(0)
    a = acc_ref[0] + acc_ref[1]
    off = pl.multiple_of(i * _BP, 128)
    den = den_ref[:, pl.ds(off, _BP)]
    dsum = den[0] + den[1] + jnp.float32(1e-16)
    out_ref[...] = a / dsum[:, None] + bias_ref[...]


def _tc_combine(acc2, den2, bias):
    grid = _NPAD // _BP
    return pl.pallas_call(
        _comb_body,
        grid=(grid,),
        in_specs=[
            pl.BlockSpec((2, _BP, _D), lambda i: (0, i, 0)),
            pl.BlockSpec((2, _NPAD), lambda i: (0, 0)),
            pl.BlockSpec((1, _D), lambda i: (0, 0)),
        ],
        out_specs=pl.BlockSpec((_BP, _D), lambda i: (i, 0)),
        out_shape=jax.ShapeDtypeStruct((_NPAD, _D), jnp.float32),
    )(acc2, den2, bias)


def kernel(x, edge_index, W, att_src, att_dst, bias):
    att_s = att_src.reshape(1, _D)
    att_d = att_dst.reshape(1, _D)
    xp = jnp.pad(x, ((0, _NPAD - _N), (0, 0)))
    xw, asrc8, adst8 = _tc_proj(xp, W, att_s, att_d)
    src = edge_index[0]
    dst = edge_index[1]
    alpha, amax2 = _sc_alpha_amax(src, dst, asrc8[0], adst8[0])
    acc2, den2 = _sc_aggregate(src, dst, alpha, amax2, xw)
    out = _tc_combine(acc2, den2, bias.reshape(1, _D))
    return out[:_N]


# speedup vs baseline: 39.3278x; 1.6019x over previous
import functools

import jax
import jax.numpy as jnp
from jax import lax
from jax.experimental import pallas as pl
from jax.experimental.pallas import tpu as pltpu
from jax.experimental.pallas import tpu_sc as plsc

_N = 10000
_E = 320000
_D = 128
_NPAD = 10240
_NW = 32
_EPT = _E // _NW
_B1 = 400
_C = 80
_SL = _NPAD // 16
_CH = 1280

_mesh = plsc.VectorSubcoreMesh(core_axis_name="c", subcore_axis_name="s")


_BP = 512


def _proj_body(x_ref, w_ref, as_ref, ad_ref, xw_ref, asrc_ref, adst_ref):
    xw = jnp.dot(x_ref[...], w_ref[...], preferred_element_type=jnp.float32)
    xw_ref[...] = xw
    a_s = jnp.sum(xw * as_ref[...], axis=1)
    a_d = jnp.sum(xw * ad_ref[...], axis=1)
    asrc_ref[...] = jnp.broadcast_to(a_s[None, :], (8, _BP))
    adst_ref[...] = jnp.broadcast_to(a_d[None, :], (8, _BP))


def _tc_proj(x, w, att_s, att_d):
    grid = _NPAD // _BP
    return pl.pallas_call(
        _proj_body,
        grid=(grid,),
        in_specs=[
            pl.BlockSpec((_BP, _D), lambda i: (i, 0)),
            pl.BlockSpec((_D, _D), lambda i: (0, 0)),
            pl.BlockSpec((1, _D), lambda i: (0, 0)),
            pl.BlockSpec((1, _D), lambda i: (0, 0)),
        ],
        out_specs=[
            pl.BlockSpec((_BP, _D), lambda i: (i, 0)),
            pl.BlockSpec((8, _BP), lambda i: (0, i)),
            pl.BlockSpec((8, _BP), lambda i: (0, i)),
        ],
        out_shape=[
            jax.ShapeDtypeStruct((_NPAD, _D), jnp.float32),
            jax.ShapeDtypeStruct((8, _NPAD), jnp.float32),
            jax.ShapeDtypeStruct((8, _NPAD), jnp.float32),
        ],
    )(x, w, att_s, att_d)


@functools.partial(
    pl.kernel,
    mesh=_mesh,
    compiler_params=pltpu.CompilerParams(needs_layout_passes=False),
    out_type=[
        jax.ShapeDtypeStruct((_E,), jnp.float32),
        jax.ShapeDtypeStruct((2, _NPAD), jnp.float32),
    ],
    scratch_types=[
        pltpu.VMEM((_NPAD,), jnp.float32),
        pltpu.VMEM((_NPAD,), jnp.float32),
        pltpu.VMEM((_NPAD,), jnp.float32),
        pltpu.VMEM((_B1,), jnp.int32),
        pltpu.VMEM((_B1,), jnp.int32),
        pltpu.VMEM((_B1,), jnp.float32),
        pltpu.VMEM_SHARED((16, _NPAD), jnp.float32),
        pltpu.VMEM((16, _SL), jnp.float32),
    ],
)
def _sc_alpha_amax(src_h, dst_h, asrc_h, adst_h, alpha_h, amax_h,
                   asrc_v, adst_v, amax_v, src_v, dst_v, al_v, stage_sh, red_v):
    cid = lax.axis_index("c")
    sid = lax.axis_index("s")
    wid = cid * 16 + sid

    pltpu.sync_copy(asrc_h, asrc_v)
    pltpu.sync_copy(adst_h, adst_v)

    neg_inf = jnp.full((16,), -jnp.inf, jnp.float32)

    def _init(i, carry):
        amax_v[pl.ds(i * 16, 16)] = neg_inf
        return carry

    lax.fori_loop(0, _NPAD // 16, _init, 0)

    ebase = wid * _EPT

    def _blk(b, carry):
        off = ebase + b * _B1
        pltpu.sync_copy(src_h.at[pl.ds(off, _B1)], src_v)
        pltpu.sync_copy(dst_h.at[pl.ds(off, _B1)], dst_v)

        def _grp(g, c2):
            s = src_v[pl.ds(g * 16, 16)]
            d = dst_v[pl.ds(g * 16, 16)]
            al = plsc.load_gather(asrc_v, [s]) + plsc.load_gather(adst_v, [d])
            al = jnp.where(al >= 0.0, al, al * 0.2)
            al_v[pl.ds(g * 16, 16)] = al

            cur = plsc.load_gather(amax_v, [d])
            need = al > cur
            plsc.store_scatter(amax_v, [d], al, mask=need)
            chk = plsc.load_gather(amax_v, [d])
            still = jnp.where(jnp.logical_and(need, chk < al), 1, 0)

            def _cond(st):
                return jnp.max(st) > 0

            def _body(st):
                m = st > 0
                plsc.store_scatter(amax_v, [d], al, mask=m)
                c = plsc.load_gather(amax_v, [d])
                return jnp.where(jnp.logical_and(m, c < al), 1, 0)

            lax.while_loop(_cond, _body, still)
            return c2

        lax.fori_loop(0, _B1 // 16, _grp, 0)
        pltpu.sync_copy(al_v, alpha_h.at[pl.ds(off, _B1)])
        return carry

    lax.fori_loop(0, _EPT // _B1, _blk, 0)

    pltpu.sync_copy(amax_v, stage_sh.at[sid])
    plsc.subcore_barrier()
    colbase = sid * _SL
    for r in range(16):
        pltpu.sync_copy(stage_sh.at[r, pl.ds(colbase, _SL)], red_v.at[r])

    def _red(i, carry):
        m = red_v[0, pl.ds(i * 16, 16)]
        for r in range(1, 16):
            m = jnp.maximum(m, red_v[r, pl.ds(i * 16, 16)])
        amax_v[pl.ds(i * 16, 16)] = m
        return carry

    lax.fori_loop(0, _SL // 16, _red, 0)
    pltpu.sync_copy(amax_v.at[pl.ds(0, _SL)], amax_h.at[cid, pl.ds(colbase, _SL)])


@functools.partial(
    pl.kernel,
    mesh=_mesh,
    compiler_params=pltpu.CompilerParams(needs_layout_passes=False),
    out_type=[
        jax.ShapeDtypeStruct((2, _NPAD, _D), jnp.float32),
        jax.ShapeDtypeStruct((2, _NPAD), jnp.float32),
    ],
    scratch_types=[
        pltpu.VMEM((_NPAD,), jnp.float32),
        pltpu.VMEM((_CH,), jnp.float32),
        pltpu.VMEM((_CH,), jnp.float32),
        [pltpu.VMEM((_C,), jnp.int32) for _ in range(3)],
        [pltpu.VMEM((_C,), jnp.int32) for _ in range(3)],
        [pltpu.VMEM((_C,), jnp.float32) for _ in range(3)],
        [pltpu.VMEM((_C,), jnp.float32) for _ in range(3)],
        [pltpu.VMEM((_C,), jnp.int32) for _ in range(3)],
        [pltpu.VMEM((_C, _D), jnp.float32) for _ in range(3)],
        pltpu.VMEM_SHARED((_NPAD, _D), jnp.float32),
        pltpu.VMEM_SHARED((_NPAD,), jnp.float32),
        [pltpu.SemaphoreType.DMA for _ in range(3)],
        [pltpu.SemaphoreType.DMA for _ in range(3)],
        [pltpu.SemaphoreType.DMA for _ in range(3)],
        [pltpu.SemaphoreType.DMA for _ in range(3)],
    ],
)
def _sc_aggregate(src_h, dst_h, alpha_h, amax_h, xw_h, acc_out, den_out,
                  amf_v, tmp0_v, tmp1_v, srcb, dstb, alb, pb, sdst, rows,
                  acc_sh, den_sh, isem, gsem, ssem, dsem):
    cid = lax.axis_index("c")
    sid = lax.axis_index("s")
    wid = cid * 16 + sid
    rowbase = sid * _SL

    def _amfc(c, carry):
        off = c * _CH
        pltpu.sync_copy(amax_h.at[0, pl.ds(off, _CH)], tmp0_v)
        pltpu.sync_copy(amax_h.at[1, pl.ds(off, _CH)], tmp1_v)

        def _m(i, c2):
            amf_v[pl.ds(off + i * 16, 16)] = jnp.maximum(
                tmp0_v[pl.ds(i * 16, 16)], tmp1_v[pl.ds(i * 16, 16)])
            return c2

        lax.fori_loop(0, _CH // 16, _m, 0)
        return carry

    lax.fori_loop(0, _NPAD // _CH, _amfc, 0)

    zero16 = jnp.zeros((16,), jnp.float32)

    def _zr(i, carry):
        for j in range(_D // 16):
            rows[0][i, pl.ds(j * 16, 16)] = zero16
        return carry

    lax.fori_loop(0, _C, _zr, 0)

    def _zd(i, carry):
        pb[0][pl.ds(i * 16, 16)] = zero16
        return carry

    lax.fori_loop(0, _C // 16, _zd, 0)

    for k in range(_SL // _C):
        pltpu.sync_copy(rows[0], acc_sh.at[pl.ds(rowbase + k * _C, _C)])
        pltpu.sync_copy(pb[0], den_sh.at[pl.ds(rowbase + k * _C, _C)])
    plsc.subcore_barrier()

    ebase = wid * _EPT

    def _idx_start(j, t):
        off = ebase + j * _C
        pltpu.async_copy(src_h.at[pl.ds(off, _C)], srcb[t], isem[t])
        pltpu.async_copy(dst_h.at[pl.ds(off, _C)], dstb[t], isem[t])
        pltpu.async_copy(alpha_h.at[pl.ds(off, _C)], alb[t], isem[t])

    def _idx_wait(t):
        pltpu.make_async_copy(src_h.at[pl.ds(0, _C)], srcb[t], isem[t]).wait()
        pltpu.make_async_copy(dst_h.at[pl.ds(0, _C)], dstb[t], isem[t]).wait()
        pltpu.make_async_copy(alpha_h.at[pl.ds(0, _C)], alb[t], isem[t]).wait()

    def _scatter_wait(t):
        pltpu.make_async_copy(rows[t], acc_sh.at[sdst[t]], ssem[t]).wait()
        pltpu.make_async_copy(pb[t], den_sh.at[sdst[t]], dsem[t]).wait()

    def _half(j, t, wait_sc, do_idx, do_next):
        t1 = (t + 1) % 3
        t2 = (t + 2) % 3
        if do_idx:
            _idx_start(j + 2, t2)
        if do_next:
            if wait_sc:
                _scatter_wait(t1)
            _idx_wait(t1)
            pltpu.async_copy(xw_h.at[srcb[t1]], rows[t1], gsem[t1])
        pltpu.make_async_copy(xw_h.at[srcb[t]], rows[t], gsem[t]).wait()

        def _grp(g, c2):
            d = dstb[t][pl.ds(g * 16, 16)]
            am = plsc.load_gather(amf_v, [d])
            pb[t][pl.ds(g * 16, 16)] = jnp.exp(alb[t][pl.ds(g * 16, 16)] - am)
            sdst[t][pl.ds(g * 16, 16)] = d
            return c2

        lax.fori_loop(0, _C // 16, _grp, 0)

        @plsc.parallel_loop(0, _C, unroll=2)
        def _scale(i):
            pi = plsc.load_gather(pb[t], [jnp.full((16,), i, jnp.int32)])
            for jj in range(_D // 16):
                rows[t][i, pl.ds(jj * 16, 16)] = (
                    rows[t][i, pl.ds(jj * 16, 16)] * pi)

        pltpu.async_copy(rows[t], acc_sh.at[sdst[t]], ssem[t], add=True)
        pltpu.async_copy(pb[t], den_sh.at[sdst[t]], dsem[t], add=True)

    _idx_start(0, 0)
    _idx_start(1, 1)
    _idx_wait(0)
    pltpu.async_copy(xw_h.at[srcb[0]], rows[0], gsem[0])
    _half(0, 0, False, True, True)
    _half(1, 1, False, True, True)

    def _steady(i, carry):
        j = 2 + 3 * i
        _half(j, 2, True, True, True)
        _half(j + 1, 0, True, True, True)
        _half(j + 2, 1, True, True, True)
        return carry

    lax.fori_loop(0, 40, _steady, 0)
    _half(122, 2, True, True, True)
    _half(123, 0, True, False, True)
    _half(124, 1, True, False, False)
    _scatter_wait(2)
    _scatter_wait(0)
    _scatter_wait(1)
    plsc.subcore_barrier()

    pltpu.sync_copy(acc_sh.at[pl.ds(rowbase, _SL)],
                    acc_out.at[cid, pl.ds(rowbase, _SL)])
    pltpu.sync_copy(den_sh.at[pl.ds(rowbase, _SL)],
                    den_out.at[cid, pl.ds(rowbase, _SL)])


def _comb_body(acc_ref, den_ref, bias_ref, out_ref):
    i = pl.program_id(0)
    a = acc_ref[0] + acc_ref[1]
    off = pl.multiple_of(i * _BP, 128)
    den = den_ref[:, pl.ds(off, _BP)]
    dsum = den[0] + den[1] + jnp.float32(1e-16)
    out_ref[...] = a / dsum[:, None] + bias_ref[...]


def _tc_combine(acc2, den2, bias):
    grid = _NPAD // _BP
    return pl.pallas_call(
        _comb_body,
        grid=(grid,),
        in_specs=[
            pl.BlockSpec((2, _BP, _D), lambda i: (0, i, 0)),
            pl.BlockSpec((2, _NPAD), lambda i: (0, 0)),
            pl.BlockSpec((1, _D), lambda i: (0, 0)),
        ],
        out_specs=pl.BlockSpec((_BP, _D), lambda i: (i, 0)),
        out_shape=jax.ShapeDtypeStruct((_NPAD, _D), jnp.float32),
    )(acc2, den2, bias)


def kernel(x, edge_index, W, att_src, att_dst, bias):
    att_s = att_src.reshape(1, _D)
    att_d = att_dst.reshape(1, _D)
    xp = jnp.pad(x, ((0, _NPAD - _N), (0, 0)))
    xw, asrc8, adst8 = _tc_proj(xp, W, att_s, att_d)
    src = edge_index[0]
    dst = edge_index[1]
    alpha, amax2 = _sc_alpha_amax(src, dst, asrc8[0], adst8[0])
    acc2, den2 = _sc_aggregate(src, dst, alpha, amax2, xw)
    out = _tc_combine(acc2, den2, bias.reshape(1, _D))
    return out[:_N]


# speedup vs baseline: 46.1690x; 1.1740x over previous
import functools

import jax
import jax.numpy as jnp
from jax import lax
from jax.experimental import pallas as pl
from jax.experimental.pallas import tpu as pltpu
from jax.experimental.pallas import tpu_sc as plsc

_N = 10000
_E = 320000
_D = 128
_NPAD = 10240
_NW = 32
_EPT = _E // _NW
_B1 = 400
_C = 80
_SL = _NPAD // 16
_CH = 1280

_mesh = plsc.VectorSubcoreMesh(core_axis_name="c", subcore_axis_name="s")


_BP = 512
_BPRJ = 2048


def _proj_body(x_ref, w_ref, as_ref, ad_ref, xw_ref, asrc_ref, adst_ref):
    xw = jnp.dot(x_ref[...], w_ref[...], preferred_element_type=jnp.float32)
    xw_ref[...] = xw
    a_s = jnp.sum(xw * as_ref[...], axis=1)
    a_d = jnp.sum(xw * ad_ref[...], axis=1)
    asrc_ref[...] = jnp.broadcast_to(a_s[None, :], (8, _BPRJ))
    adst_ref[...] = jnp.broadcast_to(a_d[None, :], (8, _BPRJ))


def _tc_proj(x, w, att_s, att_d):
    grid = _NPAD // _BPRJ
    return pl.pallas_call(
        _proj_body,
        grid=(grid,),
        in_specs=[
            pl.BlockSpec((_BPRJ, _D), lambda i: (i, 0)),
            pl.BlockSpec((_D, _D), lambda i: (0, 0)),
            pl.BlockSpec((1, _D), lambda i: (0, 0)),
            pl.BlockSpec((1, _D), lambda i: (0, 0)),
        ],
        out_specs=[
            pl.BlockSpec((_BPRJ, _D), lambda i: (i, 0)),
            pl.BlockSpec((8, _BPRJ), lambda i: (0, i)),
            pl.BlockSpec((8, _BPRJ), lambda i: (0, i)),
        ],
        out_shape=[
            jax.ShapeDtypeStruct((_NPAD, _D), jnp.float32),
            jax.ShapeDtypeStruct((8, _NPAD), jnp.float32),
            jax.ShapeDtypeStruct((8, _NPAD), jnp.float32),
        ],
    )(x, w, att_s, att_d)


@functools.partial(
    pl.kernel,
    mesh=_mesh,
    compiler_params=pltpu.CompilerParams(needs_layout_passes=False),
    out_type=[
        jax.ShapeDtypeStruct((_E,), jnp.float32),
        jax.ShapeDtypeStruct((2, _NPAD), jnp.float32),
    ],
    scratch_types=[
        pltpu.VMEM((_NPAD,), jnp.float32),
        pltpu.VMEM((_NPAD,), jnp.float32),
        pltpu.VMEM((_NPAD,), jnp.float32),
        [pltpu.VMEM((_B1,), jnp.int32) for _ in range(2)],
        [pltpu.VMEM((_B1,), jnp.int32) for _ in range(2)],
        [pltpu.VMEM((_B1,), jnp.float32) for _ in range(2)],
        pltpu.VMEM_SHARED((16, _NPAD), jnp.float32),
        pltpu.VMEM((16, _SL), jnp.float32),
        [pltpu.SemaphoreType.DMA for _ in range(2)],
        [pltpu.SemaphoreType.DMA for _ in range(2)],
    ],
)
def _sc_alpha_amax(src_h, dst_h, asrc_h, adst_h, alpha_h, amax_h,
                   asrc_v, adst_v, amax_v, srcb, dstb, alb, stage_sh, red_v,
                   isem, wsem):
    cid = lax.axis_index("c")
    sid = lax.axis_index("s")
    wid = cid * 16 + sid

    pltpu.sync_copy(asrc_h.at[0], asrc_v)
    pltpu.sync_copy(adst_h.at[0], adst_v)

    neg_inf = jnp.full((16,), -jnp.inf, jnp.float32)

    def _init(i, carry):
        amax_v[pl.ds(i * 16, 16)] = neg_inf
        return carry

    lax.fori_loop(0, _NPAD // 16, _init, 0)

    ebase = wid * _EPT
    nblk = _EPT // _B1

    def _idx_start(j, t):
        off = ebase + j * _B1
        pltpu.async_copy(src_h.at[pl.ds(off, _B1)], srcb[t], isem[t])
        pltpu.async_copy(dst_h.at[pl.ds(off, _B1)], dstb[t], isem[t])

    def _idx_wait(t):
        pltpu.make_async_copy(src_h.at[pl.ds(0, _B1)], srcb[t], isem[t]).wait()
        pltpu.make_async_copy(dst_h.at[pl.ds(0, _B1)], dstb[t], isem[t]).wait()

    def _wb_wait(t):
        pltpu.make_async_copy(alb[t], alpha_h.at[pl.ds(0, _B1)], wsem[t]).wait()

    def _half(j, t, wait_wb, do_idx):
        if do_idx:
            _idx_start(j + 1, 1 - t)
        _idx_wait(t)
        if wait_wb:
            _wb_wait(t)

        def _grp(g, c2):
            s = srcb[t][pl.ds(g * 16, 16)]
            d = dstb[t][pl.ds(g * 16, 16)]
            al = plsc.load_gather(asrc_v, [s]) + plsc.load_gather(adst_v, [d])
            al = jnp.where(al >= 0.0, al, al * 0.2)
            alb[t][pl.ds(g * 16, 16)] = al

            cur = plsc.load_gather(amax_v, [d])
            need = al > cur
            plsc.store_scatter(amax_v, [d], al, mask=need)
            chk = plsc.load_gather(amax_v, [d])
            still = jnp.where(jnp.logical_and(need, chk < al), 1, 0)

            def _cond(st):
                return jnp.max(st) > 0

            def _body(st):
                m = st > 0
                plsc.store_scatter(amax_v, [d], al, mask=m)
                c = plsc.load_gather(amax_v, [d])
                return jnp.where(jnp.logical_and(m, c < al), 1, 0)

            lax.while_loop(_cond, _body, still)
            return c2

        lax.fori_loop(0, _B1 // 16, _grp, 0)
        pltpu.async_copy(alb[t], alpha_h.at[pl.ds(ebase + j * _B1, _B1)],
                         wsem[t])

    _idx_start(0, 0)
    _half(0, 0, False, True)
    _half(1, 1, False, True)

    def _steady(i, carry):
        j = 2 + 2 * i
        _half(j, 0, True, True)
        _half(j + 1, 1, True, True)
        return carry

    lax.fori_loop(0, (nblk - 3) // 2, _steady, 0)
    _half(24, 0, True, False)
    _wb_wait(1)
    _wb_wait(0)

    pltpu.sync_copy(amax_v, stage_sh.at[sid])
    plsc.subcore_barrier()
    colbase = sid * _SL
    for r in range(16):
        pltpu.sync_copy(stage_sh.at[r, pl.ds(colbase, _SL)], red_v.at[r])

    def _red(i, carry):
        m = red_v[0, pl.ds(i * 16, 16)]
        for r in range(1, 16):
            m = jnp.maximum(m, red_v[r, pl.ds(i * 16, 16)])
        amax_v[pl.ds(i * 16, 16)] = m
        return carry

    lax.fori_loop(0, _SL // 16, _red, 0)
    pltpu.sync_copy(amax_v.at[pl.ds(0, _SL)], amax_h.at[cid, pl.ds(colbase, _SL)])


@functools.partial(
    pl.kernel,
    mesh=_mesh,
    compiler_params=pltpu.CompilerParams(needs_layout_passes=False),
    out_type=[
        jax.ShapeDtypeStruct((2, _NPAD, _D), jnp.float32),
        jax.ShapeDtypeStruct((2, _NPAD), jnp.float32),
    ],
    scratch_types=[
        pltpu.VMEM((_NPAD,), jnp.float32),
        pltpu.VMEM((_CH,), jnp.float32),
        pltpu.VMEM((_CH,), jnp.float32),
        [pltpu.VMEM((_C,), jnp.int32) for _ in range(3)],
        [pltpu.VMEM((_C,), jnp.int32) for _ in range(3)],
        [pltpu.VMEM((_C,), jnp.float32) for _ in range(3)],
        [pltpu.VMEM((_C,), jnp.float32) for _ in range(3)],
        [pltpu.VMEM((_C,), jnp.int32) for _ in range(3)],
        [pltpu.VMEM((_C, _D), jnp.float32) for _ in range(3)],
        pltpu.VMEM_SHARED((_NPAD, _D), jnp.float32),
        pltpu.VMEM_SHARED((_NPAD,), jnp.float32),
        [pltpu.SemaphoreType.DMA for _ in range(3)],
        [pltpu.SemaphoreType.DMA for _ in range(3)],
        [pltpu.SemaphoreType.DMA for _ in range(3)],
        [pltpu.SemaphoreType.DMA for _ in range(3)],
    ],
)
def _sc_aggregate(src_h, dst_h, alpha_h, amax_h, xw_h, acc_out, den_out,
                  amf_v, tmp0_v, tmp1_v, srcb, dstb, alb, pb, sdst, rows,
                  acc_sh, den_sh, isem, gsem, ssem, dsem):
    cid = lax.axis_index("c")
    sid = lax.axis_index("s")
    wid = cid * 16 + sid
    rowbase = sid * _SL

    def _amfc(c, carry):
        off = c * _CH
        pltpu.sync_copy(amax_h.at[0, pl.ds(off, _CH)], tmp0_v)
        pltpu.sync_copy(amax_h.at[1, pl.ds(off, _CH)], tmp1_v)

        def _m(i, c2):
            amf_v[pl.ds(off + i * 16, 16)] = jnp.maximum(
                tmp0_v[pl.ds(i * 16, 16)], tmp1_v[pl.ds(i * 16, 16)])
            return c2

        lax.fori_loop(0, _CH // 16, _m, 0)
        return carry

    lax.fori_loop(0, _NPAD // _CH, _amfc, 0)

    zero16 = jnp.zeros((16,), jnp.float32)

    def _zr(i, carry):
        for j in range(_D // 16):
            rows[0][i, pl.ds(j * 16, 16)] = zero16
        return carry

    lax.fori_loop(0, _C, _zr, 0)

    def _zd(i, carry):
        pb[0][pl.ds(i * 16, 16)] = zero16
        return carry

    lax.fori_loop(0, _C // 16, _zd, 0)

    for k in range(_SL // _C):
        pltpu.sync_copy(rows[0], acc_sh.at[pl.ds(rowbase + k * _C, _C)])
        pltpu.sync_copy(pb[0], den_sh.at[pl.ds(rowbase + k * _C, _C)])
    plsc.subcore_barrier()

    ebase = wid * _EPT

    def _idx_start(j, t):
        off = ebase + j * _C
        pltpu.async_copy(src_h.at[pl.ds(off, _C)], srcb[t], isem[t])
        pltpu.async_copy(dst_h.at[pl.ds(off, _C)], dstb[t], isem[t])
        pltpu.async_copy(alpha_h.at[pl.ds(off, _C)], alb[t], isem[t])

    def _idx_wait(t):
        pltpu.make_async_copy(src_h.at[pl.ds(0, _C)], srcb[t], isem[t]).wait()
        pltpu.make_async_copy(dst_h.at[pl.ds(0, _C)], dstb[t], isem[t]).wait()
        pltpu.make_async_copy(alpha_h.at[pl.ds(0, _C)], alb[t], isem[t]).wait()

    def _scatter_wait(t):
        pltpu.make_async_copy(rows[t], acc_sh.at[sdst[t]], ssem[t]).wait()
        pltpu.make_async_copy(pb[t], den_sh.at[sdst[t]], dsem[t]).wait()

    def _half(j, t, wait_sc, do_idx, do_next):
        t1 = (t + 1) % 3
        t2 = (t + 2) % 3
        if do_idx:
            _idx_start(j + 2, t2)
        if do_next:
            if wait_sc:
                _scatter_wait(t1)
            _idx_wait(t1)
            pltpu.async_copy(xw_h.at[srcb[t1]], rows[t1], gsem[t1])
        pltpu.make_async_copy(xw_h.at[srcb[t]], rows[t], gsem[t]).wait()

        def _grp(g, c2):
            d = dstb[t][pl.ds(g * 16, 16)]
            am = plsc.load_gather(amf_v, [d])
            pb[t][pl.ds(g * 16, 16)] = jnp.exp(alb[t][pl.ds(g * 16, 16)] - am)
            sdst[t][pl.ds(g * 16, 16)] = d
            return c2

        lax.fori_loop(0, _C // 16, _grp, 0)

        @plsc.parallel_loop(0, _C, unroll=2)
        def _scale(i):
            pi = plsc.load_gather(pb[t], [jnp.full((16,), i, jnp.int32)])
            for jj in range(_D // 16):
                rows[t][i, pl.ds(jj * 16, 16)] = (
                    rows[t][i, pl.ds(jj * 16, 16)] * pi)

        pltpu.async_copy(rows[t], acc_sh.at[sdst[t]], ssem[t], add=True)
        pltpu.async_copy(pb[t], den_sh.at[sdst[t]], dsem[t], add=True)

    _idx_start(0, 0)
    _idx_start(1, 1)
    _idx_wait(0)
    pltpu.async_copy(xw_h.at[srcb[0]], rows[0], gsem[0])
    _half(0, 0, False, True, True)
    _half(1, 1, False, True, True)

    def _steady(i, carry):
        j = 2 + 3 * i
        _half(j, 2, True, True, True)
        _half(j + 1, 0, True, True, True)
        _half(j + 2, 1, True, True, True)
        return carry

    lax.fori_loop(0, 40, _steady, 0)
    _half(122, 2, True, True, True)
    _half(123, 0, True, False, True)
    _half(124, 1, True, False, False)
    _scatter_wait(2)
    _scatter_wait(0)
    _scatter_wait(1)
    plsc.subcore_barrier()

    pltpu.sync_copy(acc_sh.at[pl.ds(rowbase, _SL)],
                    acc_out.at[cid, pl.ds(rowbase, _SL)])
    pltpu.sync_copy(den_sh.at[pl.ds(rowbase, _SL)],
                    den_out.at[cid, pl.ds(rowbase, _SL)])


def _comb_body(acc_ref, den_ref, bias_ref, out_ref):
    i = pl.program_id(0)
    a = acc_ref[0] + acc_ref[1]
    off = pl.multiple_of(i * _BP, 128)
    den = den_ref[:, pl.ds(off, _BP)]
    dsum = den[0] + den[1] + jnp.float32(1e-16)
    out_ref[...] = a / dsum[:, None] + bias_ref[...]


def _tc_combine(acc2, den2, bias):
    grid = _NPAD // _BP
    return pl.pallas_call(
        _comb_body,
        grid=(grid,),
        in_specs=[
            pl.BlockSpec((2, _BP, _D), lambda i: (0, i, 0)),
            pl.BlockSpec((2, _NPAD), lambda i: (0, 0)),
            pl.BlockSpec((1, _D), lambda i: (0, 0)),
        ],
        out_specs=pl.BlockSpec((_BP, _D), lambda i: (i, 0)),
        out_shape=jax.ShapeDtypeStruct((_N, _D), jnp.float32),
    )(acc2, den2, bias)


def kernel(x, edge_index, W, att_src, att_dst, bias):
    att_s = att_src.reshape(1, _D)
    att_d = att_dst.reshape(1, _D)
    xw, asrc8, adst8 = _tc_proj(x, W, att_s, att_d)
    src = edge_index[0]
    dst = edge_index[1]
    alpha, amax2 = _sc_alpha_amax(src, dst, asrc8, adst8)
    acc2, den2 = _sc_aggregate(src, dst, alpha, amax2, xw)
    out = _tc_combine(acc2, den2, bias.reshape(1, _D))
    return out


# speedup vs baseline: 46.5065x; 1.0073x over previous
import functools

import jax
import jax.numpy as jnp
from jax import lax
from jax.experimental import pallas as pl
from jax.experimental.pallas import tpu as pltpu
from jax.experimental.pallas import tpu_sc as plsc

_N = 10000
_E = 320000
_D = 128
_NPAD = 10240
_NW = 32
_EPT = _E // _NW
_B1 = 400
_C = 80
_SL = _NPAD // 16
_CH = 1280

_mesh = plsc.VectorSubcoreMesh(core_axis_name="c", subcore_axis_name="s")


_BP = 512
_BPRJ = 2048


def _proj_body(x_ref, w_ref, as_ref, ad_ref, xw_ref, asrc_ref, adst_ref):
    xw = jnp.dot(x_ref[...], w_ref[...], preferred_element_type=jnp.float32)
    xw_ref[...] = xw
    a_s = jnp.sum(xw * as_ref[...], axis=1)
    a_d = jnp.sum(xw * ad_ref[...], axis=1)
    asrc_ref[...] = jnp.broadcast_to(a_s[None, :], (8, _BPRJ))
    adst_ref[...] = jnp.broadcast_to(a_d[None, :], (8, _BPRJ))


def _tc_proj(x, w, att_s, att_d):
    grid = _NPAD // _BPRJ
    return pl.pallas_call(
        _proj_body,
        grid=(grid,),
        in_specs=[
            pl.BlockSpec((_BPRJ, _D), lambda i: (i, 0)),
            pl.BlockSpec((_D, _D), lambda i: (0, 0)),
            pl.BlockSpec((1, _D), lambda i: (0, 0)),
            pl.BlockSpec((1, _D), lambda i: (0, 0)),
        ],
        out_specs=[
            pl.BlockSpec((_BPRJ, _D), lambda i: (i, 0)),
            pl.BlockSpec((8, _BPRJ), lambda i: (0, i)),
            pl.BlockSpec((8, _BPRJ), lambda i: (0, i)),
        ],
        out_shape=[
            jax.ShapeDtypeStruct((_NPAD, _D), jnp.float32),
            jax.ShapeDtypeStruct((8, _NPAD), jnp.float32),
            jax.ShapeDtypeStruct((8, _NPAD), jnp.float32),
        ],
    )(x, w, att_s, att_d)


@functools.partial(
    pl.kernel,
    mesh=_mesh,
    compiler_params=pltpu.CompilerParams(needs_layout_passes=False),
    out_type=[
        jax.ShapeDtypeStruct((_E,), jnp.float32),
        jax.ShapeDtypeStruct((2, _NPAD), jnp.float32),
    ],
    scratch_types=[
        pltpu.VMEM((_NPAD,), jnp.float32),
        pltpu.VMEM((_NPAD,), jnp.float32),
        pltpu.VMEM((_NPAD,), jnp.float32),
        [pltpu.VMEM((_B1,), jnp.int32) for _ in range(2)],
        [pltpu.VMEM((_B1,), jnp.int32) for _ in range(2)],
        [pltpu.VMEM((_B1,), jnp.float32) for _ in range(2)],
        pltpu.VMEM_SHARED((16, _NPAD), jnp.float32),
        pltpu.VMEM((16, _SL), jnp.float32),
        [pltpu.SemaphoreType.DMA for _ in range(2)],
        [pltpu.SemaphoreType.DMA for _ in range(2)],
    ],
)
def _sc_alpha_amax(src_h, dst_h, asrc_h, adst_h, alpha_h, amax_h,
                   asrc_v, adst_v, amax_v, srcb, dstb, alb, stage_sh, red_v,
                   isem, wsem):
    cid = lax.axis_index("c")
    sid = lax.axis_index("s")
    wid = cid * 16 + sid

    pltpu.sync_copy(asrc_h.at[0], asrc_v)
    pltpu.sync_copy(adst_h.at[0], adst_v)

    neg_inf = jnp.full((16,), -jnp.inf, jnp.float32)

    def _init(i, carry):
        amax_v[pl.ds(i * 16, 16)] = neg_inf
        return carry

    lax.fori_loop(0, _NPAD // 16, _init, 0)

    ebase = wid * _EPT
    nblk = _EPT // _B1

    def _idx_start(j, t):
        off = ebase + j * _B1
        pltpu.async_copy(src_h.at[pl.ds(off, _B1)], srcb[t], isem[t])
        pltpu.async_copy(dst_h.at[pl.ds(off, _B1)], dstb[t], isem[t])

    def _idx_wait(t):
        pltpu.make_async_copy(src_h.at[pl.ds(0, _B1)], srcb[t], isem[t]).wait()
        pltpu.make_async_copy(dst_h.at[pl.ds(0, _B1)], dstb[t], isem[t]).wait()

    def _wb_wait(t):
        pltpu.make_async_copy(alb[t], alpha_h.at[pl.ds(0, _B1)], wsem[t]).wait()

    def _half(j, t, wait_wb, do_idx):
        if do_idx:
            _idx_start(j + 1, 1 - t)
        _idx_wait(t)
        if wait_wb:
            _wb_wait(t)

        def _grp(g, c2):
            s = srcb[t][pl.ds(g * 16, 16)]
            d = dstb[t][pl.ds(g * 16, 16)]
            al = plsc.load_gather(asrc_v, [s]) + plsc.load_gather(adst_v, [d])
            al = jnp.where(al >= 0.0, al, al * 0.2)
            alb[t][pl.ds(g * 16, 16)] = al

            cur = plsc.load_gather(amax_v, [d])
            need = al > cur
            plsc.store_scatter(amax_v, [d], al, mask=need)
            chk = plsc.load_gather(amax_v, [d])
            still = jnp.where(jnp.logical_and(need, chk < al), 1, 0)

            def _cond(st):
                return jnp.max(st) > 0

            def _body(st):
                m = st > 0
                plsc.store_scatter(amax_v, [d], al, mask=m)
                c = plsc.load_gather(amax_v, [d])
                return jnp.where(jnp.logical_and(m, c < al), 1, 0)

            lax.while_loop(_cond, _body, still)
            return c2

        lax.fori_loop(0, _B1 // 16, _grp, 0)
        pltpu.async_copy(alb[t], alpha_h.at[pl.ds(ebase + j * _B1, _B1)],
                         wsem[t])

    _idx_start(0, 0)
    _half(0, 0, False, True)
    _half(1, 1, False, True)

    def _steady(i, carry):
        j = 2 + 2 * i
        _half(j, 0, True, True)
        _half(j + 1, 1, True, True)
        return carry

    lax.fori_loop(0, (nblk - 3) // 2, _steady, 0)
    _half(24, 0, True, False)
    _wb_wait(1)
    _wb_wait(0)

    pltpu.sync_copy(amax_v, stage_sh.at[sid])
    plsc.subcore_barrier()
    colbase = sid * _SL
    for r in range(16):
        pltpu.sync_copy(stage_sh.at[r, pl.ds(colbase, _SL)], red_v.at[r])

    def _red(i, carry):
        m = red_v[0, pl.ds(i * 16, 16)]
        for r in range(1, 16):
            m = jnp.maximum(m, red_v[r, pl.ds(i * 16, 16)])
        amax_v[pl.ds(i * 16, 16)] = m
        return carry

    lax.fori_loop(0, _SL // 16, _red, 0)
    pltpu.sync_copy(amax_v.at[pl.ds(0, _SL)], amax_h.at[cid, pl.ds(colbase, _SL)])


@functools.partial(
    pl.kernel,
    mesh=_mesh,
    compiler_params=pltpu.CompilerParams(needs_layout_passes=False),
    out_type=[
        jax.ShapeDtypeStruct((2, _NPAD, _D), jnp.float32),
        jax.ShapeDtypeStruct((2, _NPAD), jnp.float32),
    ],
    scratch_types=[
        pltpu.VMEM((_NPAD,), jnp.float32),
        pltpu.VMEM((_CH,), jnp.float32),
        pltpu.VMEM((_CH,), jnp.float32),
        [pltpu.VMEM((_C,), jnp.int32) for _ in range(3)],
        [pltpu.VMEM((_C,), jnp.int32) for _ in range(3)],
        [pltpu.VMEM((_C,), jnp.float32) for _ in range(3)],
        [pltpu.VMEM((_C,), jnp.float32) for _ in range(3)],
        [pltpu.VMEM((_C,), jnp.int32) for _ in range(3)],
        [pltpu.VMEM((_C, _D), jnp.float32) for _ in range(3)],
        pltpu.VMEM_SHARED((_NPAD, _D), jnp.float32),
        pltpu.VMEM_SHARED((_NPAD,), jnp.float32),
        [pltpu.SemaphoreType.DMA for _ in range(3)],
        [pltpu.SemaphoreType.DMA for _ in range(3)],
        [pltpu.SemaphoreType.DMA for _ in range(3)],
        [pltpu.SemaphoreType.DMA for _ in range(3)],
    ],
)
def _sc_aggregate(src_h, dst_h, alpha_h, amax_h, xw_h, acc_out, den_out,
                  amf_v, tmp0_v, tmp1_v, srcb, dstb, alb, pb, sdst, rows,
                  acc_sh, den_sh, isem, gsem, ssem, dsem):
    cid = lax.axis_index("c")
    sid = lax.axis_index("s")
    wid = cid * 16 + sid
    rowbase = sid * _SL

    def _amfc(c, carry):
        off = c * _CH
        pltpu.sync_copy(amax_h.at[0, pl.ds(off, _CH)], tmp0_v)
        pltpu.sync_copy(amax_h.at[1, pl.ds(off, _CH)], tmp1_v)

        def _m(i, c2):
            amf_v[pl.ds(off + i * 16, 16)] = jnp.maximum(
                tmp0_v[pl.ds(i * 16, 16)], tmp1_v[pl.ds(i * 16, 16)])
            return c2

        lax.fori_loop(0, _CH // 16, _m, 0)
        return carry

    lax.fori_loop(0, _NPAD // _CH, _amfc, 0)

    zero16 = jnp.zeros((16,), jnp.float32)

    def _zr(i, carry):
        for j in range(_D // 16):
            rows[0][i, pl.ds(j * 16, 16)] = zero16
        return carry

    lax.fori_loop(0, _C, _zr, 0)

    def _zd(i, carry):
        pb[0][pl.ds(i * 16, 16)] = zero16
        return carry

    lax.fori_loop(0, _C // 16, _zd, 0)

    for k in range(_SL // _C):
        pltpu.sync_copy(rows[0], acc_sh.at[pl.ds(rowbase + k * _C, _C)])
        pltpu.sync_copy(pb[0], den_sh.at[pl.ds(rowbase + k * _C, _C)])
    plsc.subcore_barrier()

    ebase = wid * _EPT

    def _idx_start(j, t):
        off = ebase + j * _C
        pltpu.async_copy(src_h.at[pl.ds(off, _C)], srcb[t], isem[t])
        pltpu.async_copy(dst_h.at[pl.ds(off, _C)], dstb[t], isem[t])
        pltpu.async_copy(alpha_h.at[pl.ds(off, _C)], alb[t], isem[t])

    def _idx_wait(t):
        pltpu.make_async_copy(src_h.at[pl.ds(0, _C)], srcb[t], isem[t]).wait()
        pltpu.make_async_copy(dst_h.at[pl.ds(0, _C)], dstb[t], isem[t]).wait()
        pltpu.make_async_copy(alpha_h.at[pl.ds(0, _C)], alb[t], isem[t]).wait()

    def _scatter_wait(t):
        pltpu.make_async_copy(rows[t], acc_sh.at[sdst[t]], ssem[t]).wait()
        pltpu.make_async_copy(pb[t], den_sh.at[sdst[t]], dsem[t]).wait()

    def _half(j, t, wait_sc, do_idx, do_next):
        t1 = (t + 1) % 3
        t2 = (t + 2) % 3
        if do_idx:
            _idx_start(j + 2, t2)
        if do_next:
            if wait_sc:
                _scatter_wait(t1)
            _idx_wait(t1)
            pltpu.async_copy(xw_h.at[srcb[t1]], rows[t1], gsem[t1])
        pltpu.make_async_copy(xw_h.at[srcb[t]], rows[t], gsem[t]).wait()

        @plsc.parallel_loop(0, _C // 16, unroll=5)
        def _grp(g):
            d = dstb[t][pl.ds(g * 16, 16)]
            am = plsc.load_gather(amf_v, [d])
            pb[t][pl.ds(g * 16, 16)] = jnp.exp(alb[t][pl.ds(g * 16, 16)] - am)
            sdst[t][pl.ds(g * 16, 16)] = d

        @plsc.parallel_loop(0, _C, unroll=4)
        def _scale(i):
            pi = plsc.load_gather(pb[t], [jnp.full((16,), i, jnp.int32)])
            for jj in range(_D // 16):
                rows[t][i, pl.ds(jj * 16, 16)] = (
                    rows[t][i, pl.ds(jj * 16, 16)] * pi)

        pltpu.async_copy(rows[t], acc_sh.at[sdst[t]], ssem[t], add=True)
        pltpu.async_copy(pb[t], den_sh.at[sdst[t]], dsem[t], add=True)

    _idx_start(0, 0)
    _idx_start(1, 1)
    _idx_wait(0)
    pltpu.async_copy(xw_h.at[srcb[0]], rows[0], gsem[0])
    _half(0, 0, False, True, True)
    _half(1, 1, False, True, True)

    def _steady(i, carry):
        j = 2 + 3 * i
        _half(j, 2, True, True, True)
        _half(j + 1, 0, True, True, True)
        _half(j + 2, 1, True, True, True)
        return carry

    lax.fori_loop(0, 40, _steady, 0)
    _half(122, 2, True, True, True)
    _half(123, 0, True, False, True)
    _half(124, 1, True, False, False)
    _scatter_wait(2)
    _scatter_wait(0)
    _scatter_wait(1)
    plsc.subcore_barrier()

    pltpu.sync_copy(acc_sh.at[pl.ds(rowbase, _SL)],
                    acc_out.at[cid, pl.ds(rowbase, _SL)])
    pltpu.sync_copy(den_sh.at[pl.ds(rowbase, _SL)],
                    den_out.at[cid, pl.ds(rowbase, _SL)])


def _comb_body(acc_ref, den_ref, bias_ref, out_ref):
    i = pl.program_id(0)
    a = acc_ref[0] + acc_ref[1]
    off = pl.multiple_of(i * _BP, 128)
    den = den_ref[:, pl.ds(off, _BP)]
    dsum = den[0] + den[1] + jnp.float32(1e-16)
    out_ref[...] = a / dsum[:, None] + bias_ref[...]


def _tc_combine(acc2, den2, bias):
    grid = _NPAD // _BP
    return pl.pallas_call(
        _comb_body,
        grid=(grid,),
        in_specs=[
            pl.BlockSpec((2, _BP, _D), lambda i: (0, i, 0)),
            pl.BlockSpec((2, _NPAD), lambda i: (0, 0)),
            pl.BlockSpec((1, _D), lambda i: (0, 0)),
        ],
        out_specs=pl.BlockSpec((_BP, _D), lambda i: (i, 0)),
        out_shape=jax.ShapeDtypeStruct((_N, _D), jnp.float32),
    )(acc2, den2, bias)


def kernel(x, edge_index, W, att_src, att_dst, bias):
    att_s = att_src.reshape(1, _D)
    att_d = att_dst.reshape(1, _D)
    xw, asrc8, adst8 = _tc_proj(x, W, att_s, att_d)
    src = edge_index[0]
    dst = edge_index[1]
    alpha, amax2 = _sc_alpha_amax(src, dst, asrc8, adst8)
    acc2, den2 = _sc_aggregate(src, dst, alpha, amax2, xw)
    out = _tc_combine(acc2, den2, bias.reshape(1, _D))
    return out


# speedup vs baseline: 47.4275x; 1.0198x over previous
import functools

import jax
import jax.numpy as jnp
from jax import lax
from jax.experimental import pallas as pl
from jax.experimental.pallas import tpu as pltpu
from jax.experimental.pallas import tpu_sc as plsc

_N = 10000
_E = 320000
_D = 128
_NPAD = 10240
_NW = 32
_EPT = _E // _NW
_B1 = 400
_C = 80
_SL = _NPAD // 16
_CH = 1280

_mesh = plsc.VectorSubcoreMesh(core_axis_name="c", subcore_axis_name="s")


_BP = 1024
_BPRJ = 2048


def _proj_body(x_ref, w_ref, as_ref, ad_ref, xw_ref, asrc_ref, adst_ref):
    xw = jnp.dot(x_ref[...], w_ref[...], preferred_element_type=jnp.float32)
    xw_ref[...] = xw
    a_s = jnp.sum(xw * as_ref[...], axis=1)
    a_d = jnp.sum(xw * ad_ref[...], axis=1)
    asrc_ref[...] = jnp.broadcast_to(a_s[None, :], (8, _BPRJ))
    adst_ref[...] = jnp.broadcast_to(a_d[None, :], (8, _BPRJ))


def _tc_proj(x, w, att_s, att_d):
    grid = _NPAD // _BPRJ
    return pl.pallas_call(
        _proj_body,
        grid=(grid,),
        in_specs=[
            pl.BlockSpec((_BPRJ, _D), lambda i: (i, 0)),
            pl.BlockSpec((_D, _D), lambda i: (0, 0)),
            pl.BlockSpec((1, _D), lambda i: (0, 0)),
            pl.BlockSpec((1, _D), lambda i: (0, 0)),
        ],
        out_specs=[
            pl.BlockSpec((_BPRJ, _D), lambda i: (i, 0)),
            pl.BlockSpec((8, _BPRJ), lambda i: (0, i)),
            pl.BlockSpec((8, _BPRJ), lambda i: (0, i)),
        ],
        out_shape=[
            jax.ShapeDtypeStruct((_NPAD, _D), jnp.float32),
            jax.ShapeDtypeStruct((8, _NPAD), jnp.float32),
            jax.ShapeDtypeStruct((8, _NPAD), jnp.float32),
        ],
    )(x, w, att_s, att_d)


@functools.partial(
    pl.kernel,
    mesh=_mesh,
    compiler_params=pltpu.CompilerParams(needs_layout_passes=False),
    out_type=[
        jax.ShapeDtypeStruct((_E,), jnp.float32),
        jax.ShapeDtypeStruct((2, _NPAD), jnp.float32),
    ],
    scratch_types=[
        pltpu.VMEM((_NPAD,), jnp.float32),
        pltpu.VMEM((_NPAD,), jnp.float32),
        pltpu.VMEM((_NPAD,), jnp.float32),
        [pltpu.VMEM((_B1,), jnp.int32) for _ in range(2)],
        [pltpu.VMEM((_B1,), jnp.int32) for _ in range(2)],
        [pltpu.VMEM((_B1,), jnp.float32) for _ in range(2)],
        pltpu.VMEM_SHARED((16, _NPAD), jnp.float32),
        pltpu.VMEM((16, _SL), jnp.float32),
        [pltpu.SemaphoreType.DMA for _ in range(2)],
        [pltpu.SemaphoreType.DMA for _ in range(2)],
    ],
)
def _sc_alpha_amax(src_h, dst_h, asrc_h, adst_h, alpha_h, amax_h,
                   asrc_v, adst_v, amax_v, srcb, dstb, alb, stage_sh, red_v,
                   isem, wsem):
    cid = lax.axis_index("c")
    sid = lax.axis_index("s")
    wid = cid * 16 + sid

    pltpu.sync_copy(asrc_h.at[0], asrc_v)
    pltpu.sync_copy(adst_h.at[0], adst_v)

    neg_inf = jnp.full((16,), -jnp.inf, jnp.float32)

    def _init(i, carry):
        amax_v[pl.ds(i * 16, 16)] = neg_inf
        return carry

    lax.fori_loop(0, _NPAD // 16, _init, 0)

    ebase = wid * _EPT
    nblk = _EPT // _B1

    def _idx_start(j, t):
        off = ebase + j * _B1
        pltpu.async_copy(src_h.at[pl.ds(off, _B1)], srcb[t], isem[t])
        pltpu.async_copy(dst_h.at[pl.ds(off, _B1)], dstb[t], isem[t])

    def _idx_wait(t):
        pltpu.make_async_copy(src_h.at[pl.ds(0, _B1)], srcb[t], isem[t]).wait()
        pltpu.make_async_copy(dst_h.at[pl.ds(0, _B1)], dstb[t], isem[t]).wait()

    def _wb_wait(t):
        pltpu.make_async_copy(alb[t], alpha_h.at[pl.ds(0, _B1)], wsem[t]).wait()

    def _half(j, t, wait_wb, do_idx):
        if do_idx:
            _idx_start(j + 1, 1 - t)
        _idx_wait(t)
        if wait_wb:
            _wb_wait(t)

        def _grp(g, c2):
            s = srcb[t][pl.ds(g * 16, 16)]
            d = dstb[t][pl.ds(g * 16, 16)]
            al = plsc.load_gather(asrc_v, [s]) + plsc.load_gather(adst_v, [d])
            al = jnp.where(al >= 0.0, al, al * 0.2)
            alb[t][pl.ds(g * 16, 16)] = al

            cur = plsc.load_gather(amax_v, [d])
            need = al > cur
            plsc.store_scatter(amax_v, [d], al, mask=need)
            chk = plsc.load_gather(amax_v, [d])
            still = jnp.where(jnp.logical_and(need, chk < al), 1, 0)

            def _cond(st):
                return jnp.max(st) > 0

            def _body(st):
                m = st > 0
                plsc.store_scatter(amax_v, [d], al, mask=m)
                c = plsc.load_gather(amax_v, [d])
                return jnp.where(jnp.logical_and(m, c < al), 1, 0)

            lax.while_loop(_cond, _body, still)
            return c2

        lax.fori_loop(0, _B1 // 16, _grp, 0)
        pltpu.async_copy(alb[t], alpha_h.at[pl.ds(ebase + j * _B1, _B1)],
                         wsem[t])

    _idx_start(0, 0)
    _half(0, 0, False, True)
    _half(1, 1, False, True)

    def _steady(i, carry):
        j = 2 + 2 * i
        _half(j, 0, True, True)
        _half(j + 1, 1, True, True)
        return carry

    lax.fori_loop(0, (nblk - 3) // 2, _steady, 0)
    _half(24, 0, True, False)
    _wb_wait(1)
    _wb_wait(0)

    pltpu.sync_copy(amax_v, stage_sh.at[sid])
    plsc.subcore_barrier()
    colbase = sid * _SL
    for r in range(16):
        pltpu.sync_copy(stage_sh.at[r, pl.ds(colbase, _SL)], red_v.at[r])

    def _red(i, carry):
        m = red_v[0, pl.ds(i * 16, 16)]
        for r in range(1, 16):
            m = jnp.maximum(m, red_v[r, pl.ds(i * 16, 16)])
        amax_v[pl.ds(i * 16, 16)] = m
        return carry

    lax.fori_loop(0, _SL // 16, _red, 0)
    pltpu.sync_copy(amax_v.at[pl.ds(0, _SL)], amax_h.at[cid, pl.ds(colbase, _SL)])


@functools.partial(
    pl.kernel,
    mesh=_mesh,
    compiler_params=pltpu.CompilerParams(needs_layout_passes=False),
    out_type=[
        jax.ShapeDtypeStruct((2, _NPAD, _D), jnp.float32),
        jax.ShapeDtypeStruct((2, _NPAD), jnp.float32),
    ],
    scratch_types=[
        pltpu.VMEM((_NPAD,), jnp.float32),
        pltpu.VMEM((_CH,), jnp.float32),
        pltpu.VMEM((_CH,), jnp.float32),
        [pltpu.VMEM((_C,), jnp.int32) for _ in range(3)],
        [pltpu.VMEM((_C,), jnp.int32) for _ in range(3)],
        [pltpu.VMEM((_C,), jnp.float32) for _ in range(3)],
        [pltpu.VMEM((_C,), jnp.float32) for _ in range(3)],
        [pltpu.VMEM((_C,), jnp.int32) for _ in range(3)],
        [pltpu.VMEM((_C, _D), jnp.float32) for _ in range(3)],
        pltpu.VMEM_SHARED((_NPAD, _D), jnp.float32),
        pltpu.VMEM_SHARED((_NPAD,), jnp.float32),
        [pltpu.SemaphoreType.DMA for _ in range(3)],
        [pltpu.SemaphoreType.DMA for _ in range(3)],
        [pltpu.SemaphoreType.DMA for _ in range(3)],
        [pltpu.SemaphoreType.DMA for _ in range(3)],
    ],
)
def _sc_aggregate(src_h, dst_h, alpha_h, amax_h, xw_h, acc_out, den_out,
                  amf_v, tmp0_v, tmp1_v, srcb, dstb, alb, pb, sdst, rows,
                  acc_sh, den_sh, isem, gsem, ssem, dsem):
    cid = lax.axis_index("c")
    sid = lax.axis_index("s")
    wid = cid * 16 + sid
    rowbase = sid * _SL

    def _amfc(c, carry):
        off = c * _CH
        pltpu.sync_copy(amax_h.at[0, pl.ds(off, _CH)], tmp0_v)
        pltpu.sync_copy(amax_h.at[1, pl.ds(off, _CH)], tmp1_v)

        def _m(i, c2):
            amf_v[pl.ds(off + i * 16, 16)] = jnp.maximum(
                tmp0_v[pl.ds(i * 16, 16)], tmp1_v[pl.ds(i * 16, 16)])
            return c2

        lax.fori_loop(0, _CH // 16, _m, 0)
        return carry

    lax.fori_loop(0, _NPAD // _CH, _amfc, 0)

    zero16 = jnp.zeros((16,), jnp.float32)

    def _zr(i, carry):
        for j in range(_D // 16):
            rows[0][i, pl.ds(j * 16, 16)] = zero16
        return carry

    lax.fori_loop(0, _C, _zr, 0)

    def _zd(i, carry):
        pb[0][pl.ds(i * 16, 16)] = zero16
        return carry

    lax.fori_loop(0, _C // 16, _zd, 0)

    for k in range(_SL // _C):
        pltpu.sync_copy(rows[0], acc_sh.at[pl.ds(rowbase + k * _C, _C)])
        pltpu.sync_copy(pb[0], den_sh.at[pl.ds(rowbase + k * _C, _C)])
    plsc.subcore_barrier()

    ebase = wid * _EPT

    def _idx_start(j, t):
        off = ebase + j * _C
        pltpu.async_copy(src_h.at[pl.ds(off, _C)], srcb[t], isem[t])
        pltpu.async_copy(dst_h.at[pl.ds(off, _C)], dstb[t], isem[t])
        pltpu.async_copy(alpha_h.at[pl.ds(off, _C)], alb[t], isem[t])

    def _idx_wait(t):
        pltpu.make_async_copy(src_h.at[pl.ds(0, _C)], srcb[t], isem[t]).wait()
        pltpu.make_async_copy(dst_h.at[pl.ds(0, _C)], dstb[t], isem[t]).wait()
        pltpu.make_async_copy(alpha_h.at[pl.ds(0, _C)], alb[t], isem[t]).wait()

    def _scatter_wait(t):
        pltpu.make_async_copy(rows[t], acc_sh.at[sdst[t]], ssem[t]).wait()
        pltpu.make_async_copy(pb[t], den_sh.at[sdst[t]], dsem[t]).wait()

    def _half(j, t, wait_sc, do_idx, do_next):
        t1 = (t + 1) % 3
        t2 = (t + 2) % 3
        if do_idx:
            _idx_start(j + 2, t2)
        if do_next:
            if wait_sc:
                _scatter_wait(t1)
            _idx_wait(t1)
            pltpu.async_copy(xw_h.at[srcb[t1]], rows[t1], gsem[t1])
        pltpu.make_async_copy(xw_h.at[srcb[t]], rows[t], gsem[t]).wait()

        @plsc.parallel_loop(0, _C // 16, unroll=5)
        def _grp(g):
            d = dstb[t][pl.ds(g * 16, 16)]
            am = plsc.load_gather(amf_v, [d])
            pb[t][pl.ds(g * 16, 16)] = jnp.exp(alb[t][pl.ds(g * 16, 16)] - am)
            sdst[t][pl.ds(g * 16, 16)] = d

        @plsc.parallel_loop(0, _C, unroll=4)
        def _scale(i):
            pi = plsc.load_gather(pb[t], [jnp.full((16,), i, jnp.int32)])
            for jj in range(_D // 16):
                rows[t][i, pl.ds(jj * 16, 16)] = (
                    rows[t][i, pl.ds(jj * 16, 16)] * pi)

        pltpu.async_copy(rows[t], acc_sh.at[sdst[t]], ssem[t], add=True)
        pltpu.async_copy(pb[t], den_sh.at[sdst[t]], dsem[t], add=True)

    _idx_start(0, 0)
    _idx_start(1, 1)
    _idx_wait(0)
    pltpu.async_copy(xw_h.at[srcb[0]], rows[0], gsem[0])
    _half(0, 0, False, True, True)
    _half(1, 1, False, True, True)

    def _steady(i, carry):
        j = 2 + 3 * i
        _half(j, 2, True, True, True)
        _half(j + 1, 0, True, True, True)
        _half(j + 2, 1, True, True, True)
        return carry

    lax.fori_loop(0, 40, _steady, 0)
    _half(122, 2, True, True, True)
    _half(123, 0, True, False, True)
    _half(124, 1, True, False, False)
    _scatter_wait(2)
    _scatter_wait(0)
    _scatter_wait(1)
    plsc.subcore_barrier()

    pltpu.sync_copy(acc_sh.at[pl.ds(rowbase, _SL)],
                    acc_out.at[cid, pl.ds(rowbase, _SL)])
    pltpu.sync_copy(den_sh.at[pl.ds(rowbase, _SL)],
                    den_out.at[cid, pl.ds(rowbase, _SL)])


def _comb_body(acc_ref, den_ref, bias_ref, out_ref):
    i = pl.program_id(0)
    a = acc_ref[0] + acc_ref[1]
    off = pl.multiple_of(i * _BP, 128)
    den = den_ref[:, pl.ds(off, _BP)]
    dsum = den[0] + den[1] + jnp.float32(1e-16)
    out_ref[...] = a / dsum[:, None] + bias_ref[...]


def _tc_combine(acc2, den2, bias):
    grid = _NPAD // _BP
    return pl.pallas_call(
        _comb_body,
        grid=(grid,),
        in_specs=[
            pl.BlockSpec((2, _BP, _D), lambda i: (0, i, 0)),
            pl.BlockSpec((2, _NPAD), lambda i: (0, 0)),
            pl.BlockSpec((1, _D), lambda i: (0, 0)),
        ],
        out_specs=pl.BlockSpec((_BP, _D), lambda i: (i, 0)),
        out_shape=jax.ShapeDtypeStruct((_N, _D), jnp.float32),
    )(acc2, den2, bias)


def kernel(x, edge_index, W, att_src, att_dst, bias):
    att_s = att_src.reshape(1, _D)
    att_d = att_dst.reshape(1, _D)
    xw, asrc8, adst8 = _tc_proj(x, W, att_s, att_d)
    src = edge_index[0]
    dst = edge_index[1]
    alpha, amax2 = _sc_alpha_amax(src, dst, asrc8, adst8)
    acc2, den2 = _sc_aggregate(src, dst, alpha, amax2, xw)
    out = _tc_combine(acc2, den2, bias.reshape(1, _D))
    return out
